# speedup vs baseline: 1.4422x; 1.4422x over previous
import jax
import jax.numpy as jnp
from jax import lax
from jax.experimental import pallas as pl
from jax.experimental.pallas import tpu as pltpu
from jax.experimental.pallas import tpu_sc as plsc

D_MODEL = 768
N_TOKENS = 4 * 4096
NC, NS = 2, 16
NW = NC * NS
BPW = N_TOKENS // NW
CHUNK = 64
NCH = BPW // CHUNK


def _make_gather():
    mesh = plsc.VectorSubcoreMesh(core_axis_name="c", subcore_axis_name="s")

    @jax.jit
    def run(tokens_flat, W_E):
        def body(tokens_hbm, table_hbm, out_hbm, idx_v, rows_v, sem):
            wid = lax.axis_index("s") * NC + lax.axis_index("c")
            base = wid * BPW
            pltpu.sync_copy(tokens_hbm.at[pl.ds(base, BPW)], idx_v)
            for i in range(NCH):
                pltpu.async_copy(
                    table_hbm.at[idx_v.at[pl.ds(i * CHUNK, CHUNK)]],
                    rows_v,
                    sem,
                ).wait()
                pltpu.sync_copy(rows_v, out_hbm.at[pl.ds(base + i * CHUNK, CHUNK)])

        kfn = pl.kernel(
            body,
            out_type=jax.ShapeDtypeStruct((N_TOKENS, D_MODEL), jnp.float32),
            mesh=mesh,
            scratch_types=[
                pltpu.VMEM((BPW,), jnp.int32),
                pltpu.VMEM((CHUNK, D_MODEL), jnp.float32),
                pltpu.SemaphoreType.DMA,
            ],
        )
        return kfn(tokens_flat, W_E)

    return run


_gather = _make_gather()


def kernel(tokens, W_E):
    B, S = tokens.shape
    tokens_flat = tokens.reshape(-1).astype(jnp.int32)
    emb = _gather(tokens_flat, W_E)
    return (tokens, emb.reshape(B, S, D_MODEL))


# speedup vs baseline: 1.5482x; 1.0735x over previous
import jax
import jax.numpy as jnp
from jax import lax
from jax.experimental import pallas as pl
from jax.experimental.pallas import tpu as pltpu
from jax.experimental.pallas import tpu_sc as plsc

D_MODEL = 768
N_TOKENS = 4 * 4096
NC, NS = 2, 16
NW = NC * NS
BPW = N_TOKENS // NW
CHUNK = 64
NCH = BPW // CHUNK


def _make_gather():
    mesh = plsc.VectorSubcoreMesh(core_axis_name="c", subcore_axis_name="s")

    @jax.jit
    def run(tokens_flat, W_E):
        def body(tokens_hbm, table_hbm, out_hbm, idx_v,
                 rows0, rows1, gsem0, gsem1, osem0, osem1):
            wid = lax.axis_index("s") * NC + lax.axis_index("c")
            base = wid * BPW
            pltpu.sync_copy(tokens_hbm.at[pl.ds(base, BPW)], idx_v)
            bufs = (rows0, rows1)
            gsems = (gsem0, gsem1)
            osems = (osem0, osem1)

            def start_gather(i):
                b = i % 2
                return pltpu.async_copy(
                    table_hbm.at[idx_v.at[pl.ds(i * CHUNK, CHUNK)]],
                    bufs[b], gsems[b])

            def start_out(i):
                b = i % 2
                return pltpu.async_copy(
                    bufs[b], out_hbm.at[pl.ds(base + i * CHUNK, CHUNK)],
                    osems[b])

            gh = {0: start_gather(0), 1: start_gather(1)}
            oh = {}
            for i in range(NCH):
                gh[i].wait()
                oh[i] = start_out(i)
                if i + 2 < NCH:
                    oh[i].wait()
                    gh[i + 2] = start_gather(i + 2)
            for i in (NCH - 2, NCH - 1):
                oh[i].wait()

        kfn = pl.kernel(
            body,
            out_type=jax.ShapeDtypeStruct((N_TOKENS, D_MODEL), jnp.float32),
            mesh=mesh,
            scratch_types=[
                pltpu.VMEM((BPW,), jnp.int32),
                pltpu.VMEM((CHUNK, D_MODEL), jnp.float32),
                pltpu.VMEM((CHUNK, D_MODEL), jnp.float32),
                pltpu.SemaphoreType.DMA,
                pltpu.SemaphoreType.DMA,
                pltpu.SemaphoreType.DMA,
                pltpu.SemaphoreType.DMA,
            ],
        )
        return kfn(tokens_flat, W_E)

    return run


_gather = _make_gather()


def kernel(tokens, W_E):
    B, S = tokens.shape
    tokens_flat = tokens.reshape(-1).astype(jnp.int32)
    emb = _gather(tokens_flat, W_E)
    return (tokens, emb.reshape(B, S, D_MODEL))


# speedup vs baseline: 1.5703x; 1.0143x over previous
import jax
import jax.numpy as jnp
from jax import lax
from jax.experimental import pallas as pl
from jax.experimental.pallas import tpu as pltpu
from jax.experimental.pallas import tpu_sc as plsc

D_MODEL = 768
N_TOKENS = 4 * 4096
NC, NS = 2, 16
NW = NC * NS
BPW = N_TOKENS // NW
CHUNK = 32
NCH = BPW // CHUNK
NBUF = 4


def _make_gather():
    mesh = plsc.VectorSubcoreMesh(core_axis_name="c", subcore_axis_name="s")

    @jax.jit
    def run(tokens_flat, W_E):
        def body(tokens_hbm, table_hbm, out_hbm, idx_v, bufs, gsems, osems):
            wid = lax.axis_index("s") * NC + lax.axis_index("c")
            base = wid * BPW
            pltpu.sync_copy(tokens_hbm.at[pl.ds(base, BPW)], idx_v)

            def start_gather(i):
                b = i % NBUF
                return pltpu.async_copy(
                    table_hbm.at[idx_v.at[pl.ds(i * CHUNK, CHUNK)]],
                    bufs[b], gsems[b])

            def start_out(i):
                b = i % NBUF
                return pltpu.async_copy(
                    bufs[b], out_hbm.at[pl.ds(base + i * CHUNK, CHUNK)],
                    osems[b])

            gh = {i: start_gather(i) for i in range(min(NBUF, NCH))}
            oh = {}
            for i in range(NCH):
                gh[i].wait()
                oh[i] = start_out(i)
                if i + NBUF < NCH:
                    oh[i].wait()
                    gh[i + NBUF] = start_gather(i + NBUF)
            for i in range(max(0, NCH - NBUF), NCH):
                oh[i].wait()

        kfn = pl.kernel(
            body,
            out_type=jax.ShapeDtypeStruct((N_TOKENS, D_MODEL), jnp.float32),
            mesh=mesh,
            scratch_types=[
                pltpu.VMEM((BPW,), jnp.int32),
                tuple(pltpu.VMEM((CHUNK, D_MODEL), jnp.float32)
                      for _ in range(NBUF)),
                tuple(pltpu.SemaphoreType.DMA for _ in range(NBUF)),
                tuple(pltpu.SemaphoreType.DMA for _ in range(NBUF)),
            ],
        )
        return kfn(tokens_flat, W_E)

    return run


_gather = _make_gather()


def kernel(tokens, W_E):
    B, S = tokens.shape
    tokens_flat = tokens.reshape(-1).astype(jnp.int32)
    emb = _gather(tokens_flat, W_E)
    return (tokens, emb.reshape(B, S, D_MODEL))
